# baseline (speedup 1.0000x reference)
import functools

import jax
import jax.numpy as jnp
from jax import lax
from jax.experimental import pallas as pl
from jax.experimental.pallas import tpu as pltpu
from jax.experimental.pallas import tpu_sc as plsc

_NBUF = 4
_TC_CHUNKS = 256
_TC_CB = 8


def _table_body(ee_ref, ec_ref, cwt_ref, out_ref):
    out_ref[...] = ee_ref[...] + jnp.dot(
        ec_ref[...], cwt_ref[...], preferred_element_type=jnp.float32
    )


def _compute_table(element_embedding, electron_config, config_weight):
    Zmax, F = element_embedding.shape
    return pl.pallas_call(
        _table_body,
        out_shape=jax.ShapeDtypeStruct((Zmax, F), jnp.float32),
    )(element_embedding, electron_config, config_weight.T)


@functools.lru_cache(maxsize=None)
def _make_sc_gather(n_rows, chunk0, n_chunks_w, ch, F, Zmax, NC, NS):
    mesh = plsc.VectorSubcoreMesh(core_axis_name="c", subcore_axis_name="s")
    nbuf = _NBUF
    n_groups = n_chunks_w // nbuf

    @functools.partial(
        pl.kernel,
        mesh=mesh,
        out_type=jax.ShapeDtypeStruct((n_rows, F), jnp.float32),
        scratch_types=[
            pltpu.VMEM((n_chunks_w, ch), jnp.int32),
            pltpu.VMEM_SHARED((Zmax, F), jnp.float32),
            pltpu.VMEM((nbuf, ch, F), jnp.float32),
        ]
        + [pltpu.SemaphoreType.DMA] * (2 * nbuf),
    )
    def gather(table_hbm, idx_hbm, out_hbm, idx_v, table_v, rows_v, *sems):
        gsem, wsem = sems[:nbuf], sems[nbuf:]
        wid = lax.axis_index("s") * NC + lax.axis_index("c")
        row0 = chunk0 + wid * n_chunks_w
        @pl.when(lax.axis_index("s") == 0)
        def _():
            pltpu.sync_copy(table_hbm, table_v)

        pltpu.sync_copy(idx_hbm.at[pl.ds(row0, n_chunks_w)], idx_v)
        plsc.subcore_barrier()

        for b in range(nbuf):
            pltpu.async_copy(table_v.at[idx_v.at[b]], rows_v.at[b], gsem[b])

        def body(g, carry):
            for b in range(nbuf):
                j = g * nbuf + b
                pltpu.make_async_copy(
                    table_v.at[idx_v.at[j]], rows_v.at[b], gsem[b]
                ).wait()
                pltpu.async_copy(
                    rows_v.at[b], out_hbm.at[pl.ds((row0 + j) * ch, ch)], wsem[b]
                )
            for b in range(nbuf):
                j = g * nbuf + b
                pltpu.make_async_copy(
                    rows_v.at[b], out_hbm.at[pl.ds((row0 + j) * ch, ch)], wsem[b]
                ).wait()

                @pl.when(g + 1 < n_groups)
                def _():
                    jn = (g + 1) * nbuf + b
                    pltpu.async_copy(
                        table_v.at[idx_v.at[jn]], rows_v.at[b], gsem[b]
                    )

            return carry

        lax.fori_loop(0, n_groups, body, 0)

    return gather


def _tc_onehot_body(zt_ref, table_ref, _scout_ref, out_ref):
    rows = zt_ref.shape[0]
    Zmax = table_ref.shape[0]
    onehot = (
        zt_ref[...] == lax.broadcasted_iota(jnp.int32, (rows, Zmax), 1)
    ).astype(jnp.bfloat16)
    t = table_ref[...]
    t_hi = t.astype(jnp.bfloat16)
    t_lo = (t - t_hi.astype(jnp.float32)).astype(jnp.bfloat16)
    out_ref[...] = jnp.dot(
        onehot, t_hi, preferred_element_type=jnp.float32
    ) + jnp.dot(onehot, t_lo, preferred_element_type=jnp.float32)


def _tc_fill(sc_out, table, idx, tc_chunks, ch, F, Zmax):
    cb = _TC_CB
    n_rows = sc_out.shape[0]
    return pl.pallas_call(
        _tc_onehot_body,
        grid=(tc_chunks // cb,),
        in_specs=[
            pl.BlockSpec((cb * ch, 1), lambda i: (i, 0)),
            pl.BlockSpec((Zmax, F), lambda i: (0, 0)),
            pl.BlockSpec(memory_space=pl.ANY),
        ],
        out_specs=pl.BlockSpec((cb * ch, F), lambda i: (i, 0)),
        out_shape=jax.ShapeDtypeStruct((n_rows, F), jnp.float32),
        input_output_aliases={2: 0},
    )(idx[:tc_chunks].reshape(tc_chunks * ch, 1), table, sc_out)


def kernel(Z, element_embedding, config_weight, electron_config):
    B, N = Z.shape
    Zmax, F = element_embedding.shape
    table = _compute_table(element_embedding, electron_config, config_weight)

    info = plsc.get_sparse_core_info()
    NC, NS = info.num_cores, info.num_subcores
    NW = NC * NS

    ch = N
    n_chunks = B
    tc_chunks = _TC_CHUNKS
    n_chunks_w = (n_chunks - tc_chunks) // NW

    idx = Z.astype(jnp.int32)
    sc_out = _make_sc_gather(
        B * N, tc_chunks, n_chunks_w, ch, F, Zmax, NC, NS
    )(table, idx)
    out = _tc_fill(sc_out, table, idx, tc_chunks, ch, F, Zmax)
    return out.reshape(B, N, F)


# speedup vs baseline: 1.4569x; 1.4569x over previous
import functools

import jax
import jax.numpy as jnp
from jax import lax
from jax.experimental import pallas as pl
from jax.experimental.pallas import tpu as pltpu
from jax.experimental.pallas import tpu_sc as plsc


def _table_body(ee_ref, ec_ref, cwt_ref, out_ref):
    out_ref[...] = ee_ref[...] + jnp.dot(
        ec_ref[...], cwt_ref[...], preferred_element_type=jnp.float32
    )


def _compute_table(element_embedding, electron_config, config_weight):
    Zmax, F = element_embedding.shape
    return pl.pallas_call(
        _table_body,
        out_shape=jax.ShapeDtypeStruct((Zmax, F), jnp.float32),
    )(element_embedding, electron_config, config_weight.T)


_NBUF = 8


@functools.lru_cache(maxsize=None)
def _make_gather(n_rows, n_chunks_w, ch, F, Zmax, NC, NS):
    mesh = plsc.VectorSubcoreMesh(core_axis_name="c", subcore_axis_name="s")
    nbuf = _NBUF
    n_groups = n_chunks_w // nbuf

    @functools.partial(
        pl.kernel,
        mesh=mesh,
        out_type=jax.ShapeDtypeStruct((n_rows, F), jnp.float32),
        scratch_types=[
            pltpu.VMEM((n_chunks_w, ch), jnp.int32),
            pltpu.VMEM_SHARED((Zmax, F), jnp.float32),
            pltpu.VMEM((nbuf, ch, F), jnp.float32),
        ]
        + [pltpu.SemaphoreType.DMA] * (2 * nbuf),
    )
    def gather(table_hbm, idx_hbm, out_hbm, idx_v, table_v, rows_v, *sems):
        gsem, wsem = sems[:nbuf], sems[nbuf:]
        wid = lax.axis_index("s") * NC + lax.axis_index("c")
        row0 = wid * n_chunks_w
        @pl.when(lax.axis_index("s") == 0)
        def _():
            pltpu.sync_copy(table_hbm, table_v)

        pltpu.sync_copy(idx_hbm.at[pl.ds(row0, n_chunks_w)], idx_v)
        plsc.subcore_barrier()

        for b in range(nbuf):
            pltpu.async_copy(table_v.at[idx_v.at[b]], rows_v.at[b], gsem[b])

        def body(g, carry):
            for b in range(nbuf):
                j = g * nbuf + b
                pltpu.make_async_copy(
                    table_v.at[idx_v.at[j]], rows_v.at[b], gsem[b]
                ).wait()
                pltpu.async_copy(
                    rows_v.at[b], out_hbm.at[pl.ds((row0 + j) * ch, ch)], wsem[b]
                )
            for b in range(nbuf):
                j = g * nbuf + b
                pltpu.make_async_copy(
                    rows_v.at[b], out_hbm.at[pl.ds((row0 + j) * ch, ch)], wsem[b]
                ).wait()

                @pl.when(g + 1 < n_groups)
                def _():
                    jn = (g + 1) * nbuf + b
                    pltpu.async_copy(
                        table_v.at[idx_v.at[jn]], rows_v.at[b], gsem[b]
                    )

            return carry

        lax.fori_loop(0, n_groups, body, 0)

    return gather


def kernel(Z, element_embedding, config_weight, electron_config):
    B, N = Z.shape
    Zmax, F = element_embedding.shape
    table = _compute_table(element_embedding, electron_config, config_weight)

    info = plsc.get_sparse_core_info()
    NC, NS = info.num_cores, info.num_subcores
    NW = NC * NS

    ch = 64
    n_chunks = B * N // ch
    n_chunks_w = n_chunks // NW

    idx = Z.astype(jnp.int32).reshape(n_chunks, ch)
    out = _make_gather(B * N, n_chunks_w, ch, F, Zmax, NC, NS)(table, idx)
    return out.reshape(B, N, F)


# speedup vs baseline: 1.5268x; 1.0480x over previous
import functools

import jax
import jax.numpy as jnp
from jax import lax
from jax.experimental import pallas as pl
from jax.experimental.pallas import tpu as pltpu
from jax.experimental.pallas import tpu_sc as plsc


def _table_body(ee_ref, ec_ref, cwt_ref, out_ref):
    out_ref[...] = ee_ref[...] + jnp.dot(
        ec_ref[...], cwt_ref[...], preferred_element_type=jnp.float32
    )


def _compute_table(element_embedding, electron_config, config_weight):
    Zmax, F = element_embedding.shape
    return pl.pallas_call(
        _table_body,
        out_shape=jax.ShapeDtypeStruct((Zmax, F), jnp.float32),
    )(element_embedding, electron_config, config_weight.T)


_NBUF = 7


@functools.lru_cache(maxsize=None)
def _make_gather(n_rows, n_chunks_w, ch, F, Zmax, NC, NS):
    mesh = plsc.VectorSubcoreMesh(core_axis_name="c", subcore_axis_name="s")
    nbuf = _NBUF

    @functools.partial(
        pl.kernel,
        mesh=mesh,
        out_type=jax.ShapeDtypeStruct((n_rows, F), jnp.float32),
        scratch_types=[
            pltpu.VMEM((n_chunks_w, ch), jnp.int32),
            pltpu.VMEM_SHARED((Zmax, F), jnp.float32),
            pltpu.VMEM((nbuf, ch, F), jnp.float32),
        ]
        + [pltpu.SemaphoreType.DMA] * (2 * nbuf),
    )
    def gather(table_hbm, idx_hbm, out_hbm, idx_v, table_v, rows_v, *sems):
        gsem, wsem = sems[:nbuf], sems[nbuf:]
        wid = lax.axis_index("s") * NC + lax.axis_index("c")
        row0 = wid * n_chunks_w
        @pl.when(lax.axis_index("s") == 0)
        def _():
            pltpu.sync_copy(table_hbm, table_v)

        pltpu.sync_copy(idx_hbm.at[pl.ds(row0, n_chunks_w)], idx_v)
        plsc.subcore_barrier()

        n_rounds = n_chunks_w // nbuf
        tail = n_chunks_w % nbuf

        for b in range(nbuf):
            pltpu.async_copy(table_v.at[idx_v.at[b]], rows_v.at[b], gsem[b])

        def body(r, carry):
            for b in range(nbuf):
                j = r * nbuf + b
                pltpu.make_async_copy(
                    table_v.at[idx_v.at[j]], rows_v.at[b], gsem[b]
                ).wait()
                pltpu.async_copy(
                    rows_v.at[b], out_hbm.at[pl.ds((row0 + j) * ch, ch)], wsem[b]
                )
            for b in range(nbuf):
                j = r * nbuf + b
                pltpu.make_async_copy(
                    rows_v.at[b], out_hbm.at[pl.ds((row0 + j) * ch, ch)], wsem[b]
                ).wait()

                @pl.when(j + nbuf < n_chunks_w)
                def _():
                    jn = j + nbuf
                    pltpu.async_copy(
                        table_v.at[idx_v.at[jn]], rows_v.at[b], gsem[b]
                    )

            return carry

        lax.fori_loop(0, n_rounds, body, 0)

        for b in range(tail):
            j = n_rounds * nbuf + b
            pltpu.make_async_copy(
                table_v.at[idx_v.at[j]], rows_v.at[b], gsem[b]
            ).wait()
            pltpu.async_copy(
                rows_v.at[b], out_hbm.at[pl.ds((row0 + j) * ch, ch)], wsem[b]
            )
        for b in range(tail):
            j = n_rounds * nbuf + b
            pltpu.make_async_copy(
                rows_v.at[b], out_hbm.at[pl.ds((row0 + j) * ch, ch)], wsem[b]
            ).wait()

    return gather


def kernel(Z, element_embedding, config_weight, electron_config):
    B, N = Z.shape
    Zmax, F = element_embedding.shape
    table = _compute_table(element_embedding, electron_config, config_weight)

    info = plsc.get_sparse_core_info()
    NC, NS = info.num_cores, info.num_subcores
    NW = NC * NS

    ch = N
    n_chunks = B
    n_chunks_w = n_chunks // NW

    idx = Z.astype(jnp.int32)
    out = _make_gather(B * N, n_chunks_w, ch, F, Zmax, NC, NS)(table, idx)
    return out.reshape(B, N, F)
